# baseline (speedup 1.0000x reference)
import jax
import jax.numpy as jnp
from jax import lax
from jax.experimental import pallas as pl
from jax.experimental.pallas import tpu as pltpu
from jax.experimental.pallas import tpu_sc as plsc

VOCAB = 1000000
EMB = 32
B_TOK = 4096
T_TOK = 200
B = B_TOK * T_TOK

_info = plsc.get_sparse_core_info()
NC = _info.num_cores
NS = _info.num_subcores
NW = NC * NS

W_COLS = 512
U_PER_W = 61
UNIT_WORDS = W_COLS * EMB


def _detile_kernel(tab_t, tail_lin, lin, in0, in1, out0, out1, r_sems, w_sems):
    wid = lax.axis_index("s") * NC + lax.axis_index("c")
    u0 = wid * U_PER_W
    iota32 = lax.iota(jnp.int32, 16) * EMB

    in_bufs = (in0, in1)
    out_bufs = (out0, out1)

    def read(u, b):
        return pltpu.async_copy(
            tab_t.at[:, pl.ds(u * W_COLS, W_COLS)], in_bufs[b], r_sems[b])

    def wait_read(b):
        pltpu.make_async_copy(
            tab_t.at[:, pl.ds(0, W_COLS)], in_bufs[b], r_sems[b]).wait()

    def write(u, b):
        return pltpu.async_copy(
            out_bufs[b], lin.at[pl.ds(u * UNIT_WORDS, UNIT_WORDS)], w_sems[b])

    def wait_write(b):
        pltpu.make_async_copy(
            out_bufs[b], lin.at[pl.ds(0, UNIT_WORDS)], w_sems[b]).wait()

    iota = lax.iota(jnp.int32, 16)
    cmods = [lax.rem(c0 + iota, jnp.int32(EMB)) for c0 in range(EMB)]

    def transpose_unit(b):
        ib, ob = in_bufs[b], out_bufs[b]

        def blk_body(i, carry):
            bvec = i * 16 + iota
            bvec32 = bvec * EMB
            for c0 in range(EMB):
                v = plsc.load_gather(ib, [cmods[c0], bvec])
                plsc.store_scatter(ob, [bvec32 + cmods[c0]], v)
            return carry

        lax.fori_loop(0, W_COLS // 16, blk_body, 0)

    read(u0, 0)
    def pair_body(k, carry):
        u = u0 + 2 * k
        wait_read(0)
        read(u + 1, 1)
        @pl.when(k > 0)
        def _():
            wait_write(0)
        transpose_unit(0)
        write(u, 0)
        wait_read(1)
        read(u + 2, 0)
        @pl.when(k > 0)
        def _():
            wait_write(1)
        transpose_unit(1)
        write(u + 1, 1)
        return carry

    lax.fori_loop(0, (U_PER_W - 1) // 2, pair_body, 0)

    wait_read(0)
    wait_write(0)
    transpose_unit(0)
    write(u0 + U_PER_W - 1, 0)

    @pl.when(wid == 0)
    def _():
        ue = NW * U_PER_W
        pltpu.sync_copy(tab_t.at[:, pl.ds(ue * W_COLS, W_COLS)], in_bufs[1])
        wait_write(1)
        transpose_unit(1)
        pltpu.sync_copy(out_bufs[1], lin.at[pl.ds(ue * UNIT_WORDS, UNIT_WORDS)])

    @pl.when(wid == 1)
    def _():
        tail0 = (NW * U_PER_W + 1) * W_COLS
        wait_write(1)
        pltpu.sync_copy(tail_lin, out_bufs[1].at[pl.ds(0, 64 * EMB)])
        pltpu.sync_copy(out_bufs[1].at[pl.ds(0, 64 * EMB)],
                        lin.at[pl.ds(tail0 * EMB, 64 * EMB)])

    wait_write(0)
    @pl.when(wid >= 2)
    def _():
        wait_write(1)


GBLK = 4
C_STAGES = 50


def _gather_retile_kernel(table_hbm, idx_hbm, x_out, idx_v, r0, r1, ob0, ob1,
                          g_sems, w_sems):
    wid = lax.axis_index("s") * NC + lax.axis_index("c")
    u0 = wid * 200
    iota = lax.iota(jnp.int32, 16)
    cmods = [lax.rem(c0 + iota, jnp.int32(EMB)) for c0 in range(EMB)]
    crows = [lax.shift_left(lax.shift_right_logical(cm, 3), 5)
             + lax.bitwise_and(cm, 7) for cm in cmods]

    pltpu.sync_copy(idx_hbm.at[pl.ds(wid * 25600, 25600)], idx_v)

    rows = (r0, r1)
    obufs = (ob0, ob1)

    def gather(st, p):
        return pltpu.async_copy(
            table_hbm.at[idx_v.at[pl.ds(st * 512, 512)]], rows[p], g_sems[p])

    def wait_gather(p):
        pltpu.make_async_copy(
            table_hbm.at[pl.ds(0, GBLK * 128), :], rows[p], g_sems[p]).wait()

    def write(st, p):
        u = u0 + st * GBLK
        s = u // 32
        j = lax.rem(u, 32)
        row0 = s * 1024 + j * 8
        for g in range(4):
            pltpu.async_copy(
                obufs[p].at[pl.ds(g * 32, 32), :],
                x_out.at[pl.ds(row0 + g * 256, 32), :],
                w_sems[p])

    def wait_write(p):
        for _ in range(4):
            pltpu.make_async_copy(
                obufs[p].at[pl.ds(0, 32), :], x_out.at[pl.ds(0, 32), :],
                w_sems[p]).wait()

    def transpose_stage(p):
        rb, ob = rows[p], obufs[p]

        def body(i, carry):
            bvec = i * 16 + iota
            for q in range(GBLK):
                src_row = q * 128 + bvec
                for c0 in range(EMB):
                    v = plsc.load_gather(rb, [src_row, cmods[c0]])
                    plsc.store_scatter(ob, [crows[c0] + q * 8, bvec], v)
            return carry

        lax.fori_loop(0, 8, body, 0)

    gather(0, 0)
    gather(1, 1)

    def pair_body(k, carry):
        for p in range(2):
            st = 2 * k + p
            wait_gather(p)
            @pl.when(k > 0)
            def _():
                wait_write(p)
            transpose_stage(p)
            write(st, p)
            @pl.when(k < C_STAGES // 2 - 1)
            def _():
                gather(st + 2, p)
        return carry

    lax.fori_loop(0, C_STAGES // 2, pair_body, 0)
    wait_write(0)
    wait_write(1)


@jax.jit
def kernel(tokens, table):
    mesh = plsc.VectorSubcoreMesh(core_axis_name="c", subcore_axis_name="s")

    lin = pl.kernel(
        _detile_kernel,
        out_type=jax.ShapeDtypeStruct((VOCAB * EMB,), jnp.float32),
        mesh=mesh,
        scratch_types=[
            pltpu.VMEM((EMB, W_COLS), jnp.float32),
            pltpu.VMEM((EMB, W_COLS), jnp.float32),
            pltpu.VMEM((UNIT_WORDS,), jnp.float32),
            pltpu.VMEM((UNIT_WORDS,), jnp.float32),
            [pltpu.SemaphoreType.DMA, pltpu.SemaphoreType.DMA],
            [pltpu.SemaphoreType.DMA, pltpu.SemaphoreType.DMA],
        ],
        compiler_params=pltpu.CompilerParams(
            use_tc_tiling_on_sc=True, needs_layout_passes=False),
    )(table.T, table[VOCAB - 64:].reshape(64 * EMB))

    idx = tokens.T.reshape(B)
    x = pl.kernel(
        _gather_retile_kernel,
        out_type=jax.ShapeDtypeStruct((T_TOK * 1024, 128), jnp.float32),
        mesh=mesh,
        scratch_types=[
            pltpu.VMEM((B // NW,), jnp.int32),
            pltpu.VMEM((GBLK * 128, EMB), jnp.float32),
            pltpu.VMEM((GBLK * 128, EMB), jnp.float32),
            pltpu.VMEM((4 * GBLK * 8, 128), jnp.float32),
            pltpu.VMEM((4 * GBLK * 8, 128), jnp.float32),
            [pltpu.SemaphoreType.DMA, pltpu.SemaphoreType.DMA],
            [pltpu.SemaphoreType.DMA, pltpu.SemaphoreType.DMA],
        ],
        compiler_params=pltpu.CompilerParams(
            use_tc_tiling_on_sc=False, needs_layout_passes=False),
    )(lin.reshape(VOCAB, EMB), idx)
    x5 = x.reshape(T_TOK, 4, 32, 8, 128)
    return x5.transpose(2, 4, 0, 1, 3).reshape(B_TOK, T_TOK, EMB)


# speedup vs baseline: 1.0069x; 1.0069x over previous
import jax
import jax.numpy as jnp
from jax import lax
from jax.experimental import pallas as pl
from jax.experimental.pallas import tpu as pltpu
from jax.experimental.pallas import tpu_sc as plsc

VOCAB = 1000000
EMB = 32
B_TOK = 4096
T_TOK = 200
B = B_TOK * T_TOK

_info = plsc.get_sparse_core_info()
NC = _info.num_cores
NS = _info.num_subcores
NW = NC * NS

W_COLS = 512
U_PER_W = 61
UNIT_WORDS = W_COLS * EMB


def _detile_kernel(tab_t, tail_lin, lin, in0, in1, out0, out1, r_sems, w_sems):
    wid = lax.axis_index("s") * NC + lax.axis_index("c")
    u0 = wid * U_PER_W
    iota32 = lax.iota(jnp.int32, 16) * EMB

    in_bufs = (in0, in1)
    out_bufs = (out0, out1)

    def read(u, b):
        return pltpu.async_copy(
            tab_t.at[:, pl.ds(u * W_COLS, W_COLS)], in_bufs[b], r_sems[b])

    def wait_read(b):
        pltpu.make_async_copy(
            tab_t.at[:, pl.ds(0, W_COLS)], in_bufs[b], r_sems[b]).wait()

    def write(u, b):
        return pltpu.async_copy(
            out_bufs[b], lin.at[pl.ds(u * UNIT_WORDS, UNIT_WORDS)], w_sems[b])

    def wait_write(b):
        pltpu.make_async_copy(
            out_bufs[b], lin.at[pl.ds(0, UNIT_WORDS)], w_sems[b]).wait()

    iota = lax.iota(jnp.int32, 16)
    cmods = [lax.rem(c0 + iota, jnp.int32(EMB)) for c0 in range(EMB)]

    def transpose_unit(b):
        ib, ob = in_bufs[b], out_bufs[b]

        def blk_body(i, carry):
            bvec = i * 16 + iota
            bvec32 = bvec * EMB
            for c0 in range(EMB):
                v = plsc.load_gather(ib, [cmods[c0], bvec])
                plsc.store_scatter(ob, [bvec32 + cmods[c0]], v)
            return carry

        lax.fori_loop(0, W_COLS // 16, blk_body, 0)

    read(u0, 0)
    def pair_body(k, carry):
        u = u0 + 2 * k
        wait_read(0)
        read(u + 1, 1)
        @pl.when(k > 0)
        def _():
            wait_write(0)
        transpose_unit(0)
        write(u, 0)
        wait_read(1)
        read(u + 2, 0)
        @pl.when(k > 0)
        def _():
            wait_write(1)
        transpose_unit(1)
        write(u + 1, 1)
        return carry

    lax.fori_loop(0, (U_PER_W - 1) // 2, pair_body, 0)

    wait_read(0)
    wait_write(0)
    transpose_unit(0)
    write(u0 + U_PER_W - 1, 0)

    @pl.when(wid == 0)
    def _():
        ue = NW * U_PER_W
        pltpu.sync_copy(tab_t.at[:, pl.ds(ue * W_COLS, W_COLS)], in_bufs[1])
        wait_write(1)
        transpose_unit(1)
        pltpu.sync_copy(out_bufs[1], lin.at[pl.ds(ue * UNIT_WORDS, UNIT_WORDS)])

    @pl.when(wid == 1)
    def _():
        tail0 = (NW * U_PER_W + 1) * W_COLS
        wait_write(1)
        pltpu.sync_copy(tail_lin, out_bufs[1].at[pl.ds(0, 64 * EMB)])
        pltpu.sync_copy(out_bufs[1].at[pl.ds(0, 64 * EMB)],
                        lin.at[pl.ds(tail0 * EMB, 64 * EMB)])

    wait_write(0)
    @pl.when(wid >= 2)
    def _():
        wait_write(1)


C_UNITS_PER_W = 200


def _retile_kernel(g1d, out_t, gb0, gb1, ob0, ob1, r_sems, w_sems):
    wid = lax.axis_index("s") * NC + lax.axis_index("c")
    u0 = wid * C_UNITS_PER_W
    iota = lax.iota(jnp.int32, 16)
    cmods = [lax.rem(c0 + iota, jnp.int32(EMB)) for c0 in range(EMB)]

    g_bufs = (gb0, gb1)
    o_bufs = (ob0, ob1)

    def read(u, b):
        s = u // 32
        j = lax.rem(u, 32)
        return pltpu.async_copy(
            g1d.at[pl.ds((s * 4096 + j * 128) * EMB, 128 * EMB)],
            g_bufs[b], r_sems[b])

    def wait_read(b):
        pltpu.make_async_copy(
            g1d.at[pl.ds(0, 128 * EMB)], g_bufs[b], r_sems[b]).wait()

    def write(u, b):
        s = u // 32
        j = lax.rem(u, 32)
        return pltpu.async_copy(
            o_bufs[b], out_t.at[s, :, pl.ds(j * 128, 128)], w_sems[b])

    def wait_write(b):
        pltpu.make_async_copy(
            o_bufs[b], out_t.at[0, :, pl.ds(0, 128)], w_sems[b]).wait()

    def transpose_unit(b):
        gb, ob = g_bufs[b], o_bufs[b]

        def blk_body(i, carry):
            bvec = i * 16 + iota
            bvec32 = bvec * EMB
            for c0 in range(EMB):
                v = plsc.load_gather(gb, [bvec32 + cmods[c0]])
                plsc.store_scatter(ob, [cmods[c0], bvec], v)
            return carry

        lax.fori_loop(0, 8, blk_body, 0)

    read(u0, 0)
    read(u0 + 1, 1)

    def pair_body(k, carry):
        u = u0 + 2 * k
        for p in range(2):
            wait_read(p)
            @pl.when(k > 0)
            def _():
                wait_write(p)
            transpose_unit(p)
            write(u + p, p)
            @pl.when(k < C_UNITS_PER_W // 2 - 1)
            def _():
                read(u + 2 + p, p)
        return carry

    lax.fori_loop(0, C_UNITS_PER_W // 2, pair_body, 0)
    wait_write(0)
    wait_write(1)


B_PER_W = B // NW
CHUNK = 1280
N_CHUNKS = B_PER_W // CHUNK


def _gather_kernel(table_hbm, idx_hbm, out_hbm, idx_v, rows_v, g_sems, s_sems):
    wid = lax.axis_index("s") * NC + lax.axis_index("c")
    base = wid * B_PER_W

    pltpu.sync_copy(idx_hbm.at[pl.ds(base, B_PER_W)], idx_v)

    def start_gather(i, b):
        return pltpu.async_copy(
            table_hbm.at[idx_v.at[pl.ds(i * CHUNK, CHUNK)]],
            rows_v.at[b],
            g_sems[b],
        )

    def start_store(i, b):
        return pltpu.async_copy(
            rows_v.at[b],
            out_hbm.at[pl.ds(base + i * CHUNK, CHUNK)],
            s_sems[b],
        )

    gathers = [None] * N_CHUNKS
    stores = [None] * N_CHUNKS
    gathers[0] = start_gather(0, 0)
    for i in range(1, N_CHUNKS):
        b = i % 2
        if i >= 2:
            stores[i - 2].wait()
        gathers[i] = start_gather(i, b)
        gathers[i - 1].wait()
        stores[i - 1] = start_store(i - 1, 1 - b)
    gathers[N_CHUNKS - 1].wait()
    stores[N_CHUNKS - 1] = start_store(N_CHUNKS - 1, (N_CHUNKS - 1) % 2)
    stores[N_CHUNKS - 2].wait()
    stores[N_CHUNKS - 1].wait()


@jax.jit
def kernel(tokens, table):
    mesh = plsc.VectorSubcoreMesh(core_axis_name="c", subcore_axis_name="s")

    lin = pl.kernel(
        _detile_kernel,
        out_type=jax.ShapeDtypeStruct((VOCAB * EMB,), jnp.float32),
        mesh=mesh,
        scratch_types=[
            pltpu.VMEM((EMB, W_COLS), jnp.float32),
            pltpu.VMEM((EMB, W_COLS), jnp.float32),
            pltpu.VMEM((UNIT_WORDS,), jnp.float32),
            pltpu.VMEM((UNIT_WORDS,), jnp.float32),
            [pltpu.SemaphoreType.DMA, pltpu.SemaphoreType.DMA],
            [pltpu.SemaphoreType.DMA, pltpu.SemaphoreType.DMA],
        ],
        compiler_params=pltpu.CompilerParams(
            use_tc_tiling_on_sc=True, needs_layout_passes=False),
    )(table.T, table[VOCAB - 64:].reshape(64 * EMB))

    idx = tokens.T.reshape(B)
    out = pl.kernel(
        _gather_kernel,
        out_type=jax.ShapeDtypeStruct((B, EMB), jnp.float32),
        mesh=mesh,
        scratch_types=[
            pltpu.VMEM((B_PER_W,), jnp.int32),
            pltpu.VMEM((2, CHUNK, EMB), jnp.float32),
            [pltpu.SemaphoreType.DMA, pltpu.SemaphoreType.DMA],
            [pltpu.SemaphoreType.DMA, pltpu.SemaphoreType.DMA],
        ],
        compiler_params=pltpu.CompilerParams(use_tc_tiling_on_sc=False),
    )(lin.reshape(VOCAB, EMB), idx)

    out_t = pl.kernel(
        _retile_kernel,
        out_type=jax.ShapeDtypeStruct((T_TOK, EMB, B_TOK), jnp.float32),
        mesh=mesh,
        scratch_types=[
            pltpu.VMEM((128 * EMB,), jnp.float32),
            pltpu.VMEM((128 * EMB,), jnp.float32),
            pltpu.VMEM((EMB, 128), jnp.float32),
            pltpu.VMEM((EMB, 128), jnp.float32),
            [pltpu.SemaphoreType.DMA, pltpu.SemaphoreType.DMA],
            [pltpu.SemaphoreType.DMA, pltpu.SemaphoreType.DMA],
        ],
        compiler_params=pltpu.CompilerParams(
            use_tc_tiling_on_sc=True, needs_layout_passes=False),
    )(out.reshape(B * EMB))
    return out_t.transpose(2, 0, 1)
